# Initial kernel scaffold; baseline (speedup 1.0000x reference)
import functools

import jax
import jax.numpy as jnp
from jax import lax
from jax.experimental import pallas as pl
from jax.experimental.pallas import tpu as pltpu
from jax.experimental.pallas import tpu_sc as plsc

N = 100000
D = 128
NW = 32
C = 128
NCH = 25
RPW = NCH * C
NPAD = NW * RPW
LAST_FULL = (N - (NW - 1) * RPW) // C
TAIL = N - (NW - 1) * RPW - LAST_FULL * C


def _gather_body(x_hbm, idx_hbm, out_hbm, idx_v, rows_v, sem):
    wid = lax.axis_index("s") * 2 + lax.axis_index("c")
    base = wid * RPW
    pltpu.sync_copy(idx_hbm.at[pl.ds(wid * NCH, NCH)], idx_v)
    nfull = jnp.where(wid == NW - 1, LAST_FULL, NCH)

    def chunk(k, carry):
        pltpu.async_copy(x_hbm.at[idx_v.at[k]], rows_v, sem).wait()
        pltpu.sync_copy(rows_v, out_hbm.at[pl.ds(base + k * C, C)])
        return carry

    lax.fori_loop(0, nfull, chunk, 0)

    @pl.when(wid == NW - 1)
    def _():
        pltpu.async_copy(x_hbm.at[idx_v.at[LAST_FULL]], rows_v, sem).wait()
        pltpu.sync_copy(
            rows_v.at[pl.ds(0, TAIL)],
            out_hbm.at[pl.ds(base + LAST_FULL * C, TAIL)],
        )


@jax.jit
def _gather(x, idx2d):
    mesh = plsc.VectorSubcoreMesh(core_axis_name="c", subcore_axis_name="s")
    f = pl.kernel(
        _gather_body,
        out_type=jax.ShapeDtypeStruct((N, D), jnp.float32),
        mesh=mesh,
        scratch_types=[
            pltpu.VMEM((NCH, C), jnp.int32),
            pltpu.VMEM((C, D), jnp.float32),
            pltpu.SemaphoreType.DMA,
        ],
    )
    return f(x, idx2d)


def kernel(x, cell_type_indices, permutations):
    idx = permutations.reshape(-1).astype(jnp.int32)
    idx = jnp.concatenate([idx, jnp.zeros((NPAD - N,), jnp.int32)])
    return _gather(x, idx.reshape(NPAD // C, C))


# speedup vs baseline: 13.2251x; 13.2251x over previous
import functools

import jax
import jax.numpy as jnp
from jax import lax
from jax.experimental import pallas as pl
from jax.experimental.pallas import tpu as pltpu
from jax.experimental.pallas import tpu_sc as plsc

N = 100000
D = 128
NW = 32
C = 128
NCH = 25
RPW = NCH * C
NPAD = NW * RPW
LAST_FULL = (N - (NW - 1) * RPW) // C
TAIL = N - (NW - 1) * RPW - LAST_FULL * C


def _gather_body(x_hbm, idx_hbm, out_hbm, idx_v, rows_v, sem):
    wid = lax.axis_index("s") * 2 + lax.axis_index("c")
    base = pl.multiple_of(wid * RPW, RPW)
    pltpu.sync_copy(idx_hbm.at[pl.ds(base, RPW)], idx_v)
    nfull = jnp.where(wid == NW - 1, LAST_FULL, NCH)

    def chunk(k, carry):
        off = pl.multiple_of(k * C, C)
        pltpu.async_copy(x_hbm.at[idx_v.at[pl.ds(off, C)]], rows_v, sem).wait()
        pltpu.sync_copy(rows_v, out_hbm.at[pl.ds(base + off, C)])
        return carry

    lax.fori_loop(0, nfull, chunk, 0)

    @pl.when(wid == NW - 1)
    def _():
        off = LAST_FULL * C
        pltpu.async_copy(x_hbm.at[idx_v.at[pl.ds(off, C)]], rows_v, sem).wait()
        pltpu.sync_copy(
            rows_v.at[pl.ds(0, TAIL)],
            out_hbm.at[pl.ds(base + off, TAIL)],
        )


@jax.jit
def _gather(x, idx2d):
    mesh = plsc.VectorSubcoreMesh(core_axis_name="c", subcore_axis_name="s")
    f = pl.kernel(
        _gather_body,
        out_type=jax.ShapeDtypeStruct((N, D), jnp.float32),
        mesh=mesh,
        scratch_types=[
            pltpu.VMEM((RPW,), jnp.int32),
            pltpu.VMEM((C, D), jnp.float32),
            pltpu.SemaphoreType.DMA,
        ],
    )
    return f(x, idx2d)


def kernel(x, cell_type_indices, permutations):
    idx = permutations.reshape(-1).astype(jnp.int32)
    idx = jnp.concatenate([idx, jnp.zeros((NPAD - N,), jnp.int32)])
    return _gather(x, idx)
